# baseline (device time: 86113 ns/iter reference)
import jax
import jax.numpy as jnp
from jax import lax
from jax.experimental import pallas as pl
from jax.experimental.pallas import tpu as pltpu

N_DEV = 4
BLK = 128


def kernel(x):
    m, n = x.shape
    nblk = m // BLK

    def body(x_hbm, out_hbm, res_ref, in_bufs, comm_ref,
             in_sems, out_sem, send_sems, recv_sems):
        my = lax.axis_index("i")
        left = (my - 1) % N_DEV
        right = (my + 1) % N_DEV

        barrier_sem = pltpu.get_barrier_semaphore()
        for nbr in [left, right]:
            pl.semaphore_signal(
                barrier_sem, inc=1,
                device_id=(nbr,), device_id_type=pl.DeviceIdType.MESH,
            )
        pl.semaphore_wait(barrier_sem, 2)

        def in_copy(b, slot):
            return pltpu.make_async_copy(
                x_hbm.at[pl.ds(b * BLK, BLK), :],
                in_bufs.at[slot],
                in_sems.at[slot],
            )

        rows = lax.broadcasted_iota(jnp.int32, (BLK, BLK), 0)
        cols = lax.broadcasted_iota(jnp.int32, (BLK, BLK), 1)
        tri = (rows >= cols).astype(jnp.float32)

        in_copy(0, 0).start()

        def p1(b, carry):
            slot = lax.rem(b, 2)

            @pl.when(b + 1 < nblk)
            def _():
                in_copy(b + 1, lax.rem(b + 1, 2)).start()

            in_copy(b, slot).wait()
            blk = in_bufs[slot]
            cs = jnp.dot(tri, blk, preferred_element_type=jnp.float32)
            res_ref[pl.ds(b * BLK, BLK), :] = cs + carry
            return carry + cs[BLK - 1:BLK, :]

        total = lax.fori_loop(0, nblk, p1, jnp.zeros((1, n), jnp.float32))
        comm_ref[0, :, :] = total

        offset = jnp.zeros((1, n), jnp.float32)
        for h in range(N_DEV - 1):
            send_slot = h % 2
            recv_slot = (h + 1) % 2
            rdma = pltpu.make_async_remote_copy(
                src_ref=comm_ref.at[send_slot],
                dst_ref=comm_ref.at[recv_slot],
                send_sem=send_sems.at[send_slot],
                recv_sem=recv_sems.at[recv_slot],
                device_id=(right,),
                device_id_type=pl.DeviceIdType.MESH,
            )
            rdma.start()
            rdma.wait()
            origin = (my - h - 1) % N_DEV
            mask = (origin < my).astype(jnp.float32)
            offset = offset + comm_ref[recv_slot] * mask

        def out_copy(b):
            return pltpu.make_async_copy(
                res_ref.at[pl.ds(b * BLK, BLK), :],
                out_hbm.at[pl.ds(b * BLK, BLK), :],
                out_sem,
            )

        def p3(b, _):
            blk = res_ref[pl.ds(b * BLK, BLK), :]
            res_ref[pl.ds(b * BLK, BLK), :] = blk + offset
            out_copy(b).start()
            return 0

        lax.fori_loop(0, nblk, p3, 0)
        lax.fori_loop(0, nblk, lambda b, _: (out_copy(b).wait(), 0)[1], 0)

    return pl.pallas_call(
        body,
        out_shape=jax.ShapeDtypeStruct((m, n), jnp.float32),
        in_specs=[pl.BlockSpec(memory_space=pl.ANY)],
        out_specs=pl.BlockSpec(memory_space=pl.ANY),
        scratch_shapes=[
            pltpu.VMEM((m, n), jnp.float32),
            pltpu.VMEM((2, BLK, n), jnp.float32),
            pltpu.VMEM((2, 1, n), jnp.float32),
            pltpu.SemaphoreType.DMA((2,)),
            pltpu.SemaphoreType.DMA,
            pltpu.SemaphoreType.DMA((2,)),
            pltpu.SemaphoreType.DMA((2,)),
        ],
        compiler_params=pltpu.CompilerParams(
            collective_id=0, vmem_limit_bytes=60 * 1024 * 1024
        ),
    )(x)


# device time: 69572 ns/iter; 1.2378x vs baseline; 1.2378x over previous
import jax
import jax.numpy as jnp
from jax import lax
from jax.experimental import pallas as pl
from jax.experimental.pallas import tpu as pltpu

N_DEV = 4
BLK = 256


def kernel(x):
    m, n = x.shape
    nblk = m // BLK

    def body(x_hbm, out_hbm, res_ref, in_bufs, comm_ref,
             in_sems, out_sem, send_sems, recv_sems):
        my = lax.axis_index("i")
        left = (my - 1) % N_DEV
        right = (my + 1) % N_DEV

        barrier_sem = pltpu.get_barrier_semaphore()
        for nbr in [left, right]:
            pl.semaphore_signal(
                barrier_sem, inc=1,
                device_id=(nbr,), device_id_type=pl.DeviceIdType.MESH,
            )
        pl.semaphore_wait(barrier_sem, 2)

        def in_copy(b, slot):
            return pltpu.make_async_copy(
                x_hbm.at[pl.ds(b * BLK, BLK), :],
                in_bufs.at[slot],
                in_sems.at[slot],
            )

        rows = lax.broadcasted_iota(jnp.int32, (BLK, BLK), 0)
        cols = lax.broadcasted_iota(jnp.int32, (BLK, BLK), 1)
        tri = (rows >= cols).astype(jnp.float32)

        in_copy(0, 0).start()

        def p1(b, carry):
            slot = lax.rem(b, 2)

            @pl.when(b + 1 < nblk)
            def _():
                in_copy(b + 1, lax.rem(b + 1, 2)).start()

            in_copy(b, slot).wait()
            blk = in_bufs[slot]
            cs = jnp.dot(tri, blk, preferred_element_type=jnp.float32)
            res_ref[pl.ds(b * BLK, BLK), :] = cs + carry
            return carry + cs[BLK - 1:BLK, :]

        total = lax.fori_loop(0, nblk, p1, jnp.zeros((1, n), jnp.float32))
        comm_ref[0, :, :] = total

        offset = jnp.zeros((1, n), jnp.float32)
        for h in range(N_DEV - 1):
            send_slot = h % 2
            recv_slot = (h + 1) % 2
            rdma = pltpu.make_async_remote_copy(
                src_ref=comm_ref.at[send_slot],
                dst_ref=comm_ref.at[recv_slot],
                send_sem=send_sems.at[send_slot],
                recv_sem=recv_sems.at[recv_slot],
                device_id=(right,),
                device_id_type=pl.DeviceIdType.MESH,
            )
            rdma.start()
            rdma.wait()
            origin = (my - h - 1) % N_DEV
            mask = (origin < my).astype(jnp.float32)
            offset = offset + comm_ref[recv_slot] * mask

        def out_copy(b):
            return pltpu.make_async_copy(
                res_ref.at[pl.ds(b * BLK, BLK), :],
                out_hbm.at[pl.ds(b * BLK, BLK), :],
                out_sem,
            )

        def p3(b, _):
            blk = res_ref[pl.ds(b * BLK, BLK), :]
            res_ref[pl.ds(b * BLK, BLK), :] = blk + offset
            out_copy(b).start()
            return 0

        lax.fori_loop(0, nblk, p3, 0)
        lax.fori_loop(0, nblk, lambda b, _: (out_copy(b).wait(), 0)[1], 0)

    return pl.pallas_call(
        body,
        out_shape=jax.ShapeDtypeStruct((m, n), jnp.float32),
        in_specs=[pl.BlockSpec(memory_space=pl.ANY)],
        out_specs=pl.BlockSpec(memory_space=pl.ANY),
        scratch_shapes=[
            pltpu.VMEM((m, n), jnp.float32),
            pltpu.VMEM((2, BLK, n), jnp.float32),
            pltpu.VMEM((2, 1, n), jnp.float32),
            pltpu.SemaphoreType.DMA((2,)),
            pltpu.SemaphoreType.DMA,
            pltpu.SemaphoreType.DMA((2,)),
            pltpu.SemaphoreType.DMA((2,)),
        ],
        compiler_params=pltpu.CompilerParams(
            collective_id=0, vmem_limit_bytes=60 * 1024 * 1024
        ),
    )(x)


# device time: 53201 ns/iter; 1.6186x vs baseline; 1.3077x over previous
import jax
import jax.numpy as jnp
from jax import lax
from jax.experimental import pallas as pl
from jax.experimental.pallas import tpu as pltpu

N_DEV = 4
BLK = 512


def kernel(x):
    m, n = x.shape
    nblk = m // BLK

    def body(x_hbm, out_hbm, xv, comm_ref, in_sems, out_sems,
             send_sems, recv_sems):
        my = lax.axis_index("i")

        comm_ref[...] = jnp.zeros((N_DEV, 1, n), jnp.float32)

        barrier_sem = pltpu.get_barrier_semaphore()
        for j in range(N_DEV):
            @pl.when(j != my)
            def _():
                pl.semaphore_signal(
                    barrier_sem, inc=1,
                    device_id=(j,), device_id_type=pl.DeviceIdType.MESH,
                )
        pl.semaphore_wait(barrier_sem, N_DEV - 1)

        def in_copy(b):
            return pltpu.make_async_copy(
                x_hbm.at[pl.ds(b * BLK, BLK), :],
                xv.at[pl.ds(b * BLK, BLK), :],
                in_sems.at[b],
            )

        for b in range(nblk):
            in_copy(b).start()

        block_tots = []
        for b in range(nblk):
            in_copy(b).wait()
            blk = xv[pl.ds(b * BLK, BLK), :]
            block_tots.append(jnp.sum(blk, axis=0, keepdims=True))
        total = block_tots[0]
        for b in range(1, nblk):
            total = total + block_tots[b]

        comm_ref[pl.ds(my, 1)] = total[None]
        for k in range(1, N_DEV):
            @pl.when(my < k)
            def _():
                rdma = pltpu.make_async_remote_copy(
                    src_ref=comm_ref.at[my],
                    dst_ref=comm_ref.at[my],
                    send_sem=send_sems.at[k],
                    recv_sem=recv_sems.at[my],
                    device_id=(k,),
                    device_id_type=pl.DeviceIdType.MESH,
                )
                rdma.start()
                rdma.wait_send()

        for j in range(N_DEV - 1):
            @pl.when(j < my)
            def _():
                rdma = pltpu.make_async_remote_copy(
                    src_ref=comm_ref.at[j],
                    dst_ref=comm_ref.at[j],
                    send_sem=send_sems.at[j],
                    recv_sem=recv_sems.at[j],
                    device_id=(0,),
                    device_id_type=pl.DeviceIdType.MESH,
                )
                rdma.wait_recv()

        gathered = comm_ref[0] + comm_ref[1] + comm_ref[2] + comm_ref[3]
        offset = gathered - total

        rows = lax.broadcasted_iota(jnp.int32, (BLK, BLK), 0)
        cols = lax.broadcasted_iota(jnp.int32, (BLK, BLK), 1)
        tri = (rows >= cols).astype(jnp.float32)

        def out_copy(b):
            return pltpu.make_async_copy(
                xv.at[pl.ds(b * BLK, BLK), :],
                out_hbm.at[pl.ds(b * BLK, BLK), :],
                out_sems.at[b],
            )

        carry = offset
        for b in range(nblk):
            blk = xv[pl.ds(b * BLK, BLK), :]
            cs = jnp.dot(tri, blk, preferred_element_type=jnp.float32)
            xv[pl.ds(b * BLK, BLK), :] = cs + carry
            out_copy(b).start()
            carry = carry + block_tots[b]

        for b in range(nblk):
            out_copy(b).wait()

    return pl.pallas_call(
        body,
        out_shape=jax.ShapeDtypeStruct((m, n), jnp.float32),
        in_specs=[pl.BlockSpec(memory_space=pl.ANY)],
        out_specs=pl.BlockSpec(memory_space=pl.ANY),
        scratch_shapes=[
            pltpu.VMEM((m, n), jnp.float32),
            pltpu.VMEM((N_DEV, 1, n), jnp.float32),
            pltpu.SemaphoreType.DMA((nblk,)),
            pltpu.SemaphoreType.DMA((nblk,)),
            pltpu.SemaphoreType.DMA((N_DEV,)),
            pltpu.SemaphoreType.DMA((N_DEV,)),
        ],
        compiler_params=pltpu.CompilerParams(
            collective_id=0, vmem_limit_bytes=60 * 1024 * 1024
        ),
    )(x)


# device time: 52443 ns/iter; 1.6420x vs baseline; 1.0145x over previous
import jax
import jax.numpy as jnp
from jax import lax
from jax.experimental import pallas as pl
from jax.experimental.pallas import tpu as pltpu

N_DEV = 4
BLK = 512


def kernel(x):
    m, n = x.shape
    nblk = m // BLK

    def body(x_hbm, out_hbm, xv, comm_ref, in_sems, out_sems,
             send_sems, recv_sems):
        my = lax.axis_index("i")

        comm_ref[...] = jnp.zeros((N_DEV, 1, n), jnp.float32)

        barrier_sem = pltpu.get_barrier_semaphore()
        for j in range(N_DEV):
            @pl.when(j != my)
            def _():
                pl.semaphore_signal(
                    barrier_sem, inc=1,
                    device_id=(j,), device_id_type=pl.DeviceIdType.MESH,
                )
        pl.semaphore_wait(barrier_sem, N_DEV - 1)

        def in_copy(b):
            return pltpu.make_async_copy(
                x_hbm.at[pl.ds(b * BLK, BLK), :],
                xv.at[pl.ds(b * BLK, BLK), :],
                in_sems.at[b],
            )

        for b in range(nblk):
            in_copy(b).start()

        rows = lax.broadcasted_iota(jnp.int32, (BLK, BLK), 0)
        cols = lax.broadcasted_iota(jnp.int32, (BLK, BLK), 1)
        tri = (rows >= cols).astype(jnp.float32)

        carry = jnp.zeros((1, n), jnp.float32)
        for b in range(nblk):
            in_copy(b).wait()
            blk = xv[pl.ds(b * BLK, BLK), :]
            cs = jnp.dot(tri, blk, preferred_element_type=jnp.float32)
            xv[pl.ds(b * BLK, BLK), :] = cs + carry
            carry = carry + cs[BLK - 1:BLK, :]
        total = carry

        comm_ref[pl.ds(my, 1)] = total[None]
        for k in range(1, N_DEV):
            @pl.when(my < k)
            def _():
                rdma = pltpu.make_async_remote_copy(
                    src_ref=comm_ref.at[my],
                    dst_ref=comm_ref.at[my],
                    send_sem=send_sems.at[k],
                    recv_sem=recv_sems.at[my],
                    device_id=(k,),
                    device_id_type=pl.DeviceIdType.MESH,
                )
                rdma.start()
        for k in range(1, N_DEV):
            @pl.when(my < k)
            def _():
                rdma = pltpu.make_async_remote_copy(
                    src_ref=comm_ref.at[my],
                    dst_ref=comm_ref.at[my],
                    send_sem=send_sems.at[k],
                    recv_sem=recv_sems.at[my],
                    device_id=(k,),
                    device_id_type=pl.DeviceIdType.MESH,
                )
                rdma.wait_send()

        for j in range(N_DEV - 1):
            @pl.when(j < my)
            def _():
                rdma = pltpu.make_async_remote_copy(
                    src_ref=comm_ref.at[j],
                    dst_ref=comm_ref.at[j],
                    send_sem=send_sems.at[j],
                    recv_sem=recv_sems.at[j],
                    device_id=(0,),
                    device_id_type=pl.DeviceIdType.MESH,
                )
                rdma.wait_recv()

        gathered = comm_ref[0] + comm_ref[1] + comm_ref[2] + comm_ref[3]
        offset = gathered - total

        def out_copy(b):
            return pltpu.make_async_copy(
                xv.at[pl.ds(b * BLK, BLK), :],
                out_hbm.at[pl.ds(b * BLK, BLK), :],
                out_sems.at[b],
            )

        for b in range(nblk):
            blk = xv[pl.ds(b * BLK, BLK), :]
            xv[pl.ds(b * BLK, BLK), :] = blk + offset
            out_copy(b).start()

        for b in range(nblk):
            out_copy(b).wait()

    return pl.pallas_call(
        body,
        out_shape=jax.ShapeDtypeStruct((m, n), jnp.float32),
        in_specs=[pl.BlockSpec(memory_space=pl.ANY)],
        out_specs=pl.BlockSpec(memory_space=pl.ANY),
        scratch_shapes=[
            pltpu.VMEM((m, n), jnp.float32),
            pltpu.VMEM((N_DEV, 1, n), jnp.float32),
            pltpu.SemaphoreType.DMA((nblk,)),
            pltpu.SemaphoreType.DMA((nblk,)),
            pltpu.SemaphoreType.DMA((N_DEV,)),
            pltpu.SemaphoreType.DMA((N_DEV,)),
        ],
        compiler_params=pltpu.CompilerParams(
            collective_id=0, vmem_limit_bytes=60 * 1024 * 1024
        ),
    )(x)
